# baseline (speedup 1.0000x reference)
import functools

import jax
import jax.numpy as jnp
from jax import lax
from jax.experimental import pallas as pl
from jax.experimental.pallas import tpu as pltpu
from jax.experimental.pallas import tpu_sc as plsc

M = 1024
D = 8192
DP = D // 4
B = 128

NC = 2
NS = 16
NW = 16
BPW = B // NW

_MESH = plsc.VectorSubcoreMesh(core_axis_name="c", subcore_axis_name="s")


@functools.partial(
    pl.kernel,
    out_type=jax.ShapeDtypeStruct((B, DP), jnp.uint32),
    mesh=_MESH,
    scratch_types=[
        pltpu.VMEM((BPW,), jnp.int32),
        pltpu.VMEM((BPW, DP), jnp.uint32),
        pltpu.SemaphoreType.DMA,
    ],
)
def _gather_rows(masks_hbm, idx_hbm, out_hbm, idx_v, rows_v, sem):
    wid = lax.axis_index("s") * NC + lax.axis_index("c")

    @pl.when(wid < NW)
    def _():
        base = wid * BPW
        pltpu.sync_copy(idx_hbm.at[pl.ds(base, BPW)], idx_v)
        pltpu.async_copy(masks_hbm.at[idx_v], rows_v, sem).wait()
        pltpu.sync_copy(rows_v, out_hbm.at[pl.ds(base, BPW)])


def kernel(masks, idx):
    m = masks.astype(jnp.uint32)
    packed = (m[:, 0 * DP:1 * DP]
              | (m[:, 1 * DP:2 * DP] << 8)
              | (m[:, 2 * DP:3 * DP] << 16)
              | (m[:, 3 * DP:4 * DP] << 24))
    out32 = _gather_rows(packed, idx)
    return jnp.concatenate(
        [out32 & 0xFF, (out32 >> 8) & 0xFF, (out32 >> 16) & 0xFF, out32 >> 24],
        axis=1) != 0


# speedup vs baseline: 1.7722x; 1.7722x over previous
import functools

import jax
import jax.numpy as jnp
from jax import lax
from jax.experimental import pallas as pl
from jax.experimental.pallas import tpu as pltpu
from jax.experimental.pallas import tpu_sc as plsc

M = 1024
D = 8192
DP = D // 4
B = 128

NC = 2
NS = 16
NW = 16
BPW = B // NW

_MESH = plsc.VectorSubcoreMesh(core_axis_name="c", subcore_axis_name="s")


@functools.partial(
    pl.kernel,
    out_type=jax.ShapeDtypeStruct((B, DP), jnp.uint32),
    mesh=_MESH,
    compiler_params=pltpu.CompilerParams(skip_device_barrier=True),
    scratch_types=[
        pltpu.VMEM((BPW,), jnp.int32),
        pltpu.VMEM((BPW, DP), jnp.uint32),
        pltpu.SemaphoreType.DMA,
    ],
)
def _gather_rows(masks_hbm, idx_hbm, out_hbm, idx_v, rows_v, sem):
    wid = lax.axis_index("s") * NC + lax.axis_index("c")

    @pl.when(wid < NW)
    def _():
        base = wid * BPW
        pltpu.sync_copy(idx_hbm.at[pl.ds(base, BPW)], idx_v)
        pltpu.async_copy(masks_hbm.at[idx_v], rows_v, sem).wait()
        pltpu.sync_copy(rows_v, out_hbm.at[pl.ds(base, BPW)])


def kernel(masks, idx):
    packed = (masks[:, 0 * DP:1 * DP].astype(jnp.uint32)
              | (masks[:, 1 * DP:2 * DP].astype(jnp.uint32) << 8)
              | (masks[:, 2 * DP:3 * DP].astype(jnp.uint32) << 16)
              | (masks[:, 3 * DP:4 * DP].astype(jnp.uint32) << 24))
    out32 = _gather_rows(packed, idx)
    return jnp.concatenate(
        [out32 & 0xFF, (out32 >> 8) & 0xFF, (out32 >> 16) & 0xFF, out32 >> 24],
        axis=1) != 0
